# baseline (speedup 1.0000x reference)
import functools

import jax
import jax.numpy as jnp
from jax import lax
from jax.experimental import pallas as pl
from jax.experimental.pallas import tpu as pltpu
from jax.experimental.pallas import tpu_sc as plsc

TD = 1024
TF = 16
HO = 512
WO = 512
B = HO * WO
NC = 2
NS = 16
NW = NC * NS
PIX_PER_W = B // NW
CHUNK = 4096
N_CHUNKS = PIX_PER_W // CHUNK


def _prep_body(x_ref, y_ref, i00_ref, i01_ref, i10_ref, i11_ref,
               w00_ref, w01_ref, w10_ref, w11_ref):
    x = x_ref[...]
    y = y_ref[...]
    ix = jnp.clip(((x + 1.0) * TD - 1.0) * 0.5, 0.0, TD - 1.0)
    iy = jnp.clip(((y + 1.0) * TD - 1.0) * 0.5, 0.0, TD - 1.0)
    ix0 = jnp.floor(ix)
    iy0 = jnp.floor(iy)
    wx1 = ix - ix0
    wx0 = 1.0 - wx1
    wy1 = iy - iy0
    wy0 = 1.0 - wy1
    x0 = ix0.astype(jnp.int32)
    y0 = iy0.astype(jnp.int32)
    y1 = jnp.minimum(y0 + 1, TD - 1)
    x1 = jnp.minimum(x0 + 1, TD - 1)
    i00_ref[...] = y0 * TD + x0
    i01_ref[...] = y0 * TD + x1
    i10_ref[...] = y1 * TD + x0
    i11_ref[...] = y1 * TD + x1
    w00_ref[...] = wy0 * wx0
    w01_ref[...] = wy0 * wx1
    w10_ref[...] = wy1 * wx0
    w11_ref[...] = wy1 * wx1


def _prep(x, y):
    blk = pl.BlockSpec((64, WO), lambda i: (i, 0))
    shp_i = jax.ShapeDtypeStruct((HO, WO), jnp.int32)
    shp_f = jax.ShapeDtypeStruct((HO, WO), jnp.float32)
    return pl.pallas_call(
        _prep_body,
        grid=(HO // 64,),
        in_specs=[blk, blk],
        out_specs=[blk] * 8,
        out_shape=[shp_i] * 4 + [shp_f] * 4,
    )(x, y)


def _gather_body(table, i00, i01, i10, i11, r00, r01, r10, r11,
                 idx_v, rows_v, sem):
    wid = lax.axis_index("s") * NC + lax.axis_index("c")
    base = wid * PIX_PER_W
    for idx_hbm, rows_hbm in ((i00, r00), (i01, r01), (i10, r10), (i11, r11)):
        for h in range(N_CHUNKS):
            off = base + h * CHUNK
            pltpu.sync_copy(idx_hbm.at[pl.ds(off, CHUNK)], idx_v)
            pltpu.async_copy(table.at[idx_v], rows_v, sem).wait()
            pltpu.sync_copy(rows_v, rows_hbm.at[pl.ds(off, CHUNK)])


def _gather(table, i00, i01, i10, i11):
    mesh = plsc.VectorSubcoreMesh(core_axis_name="c", subcore_axis_name="s")
    shp = jax.ShapeDtypeStruct((B, TF), jnp.float32)
    k = functools.partial(
        pl.kernel,
        mesh=mesh,
        compiler_params=pltpu.CompilerParams(use_tc_tiling_on_sc=False),
        out_type=[shp] * 4,
        scratch_types=[
            pltpu.VMEM((CHUNK,), jnp.int32),
            pltpu.VMEM((CHUNK, TF), jnp.float32),
            pltpu.SemaphoreType.DMA,
        ],
    )(_gather_body)
    return k(table, i00, i01, i10, i11)


def _combine_body(w00, w01, w10, w11, r00, r01, r10, r11, out_ref):
    acc = jnp.transpose(r00[...]) * w00[...]
    acc = acc + jnp.transpose(r01[...]) * w01[...]
    acc = acc + jnp.transpose(r10[...]) * w10[...]
    acc = acc + jnp.transpose(r11[...]) * w11[...]
    out_ref[...] = acc


def _combine(w00, w01, w10, w11, r00, r01, r10, r11):
    nblk = 64
    bp = B // nblk
    wspec = pl.BlockSpec((1, bp), lambda i: (0, i))
    rspec = pl.BlockSpec((bp, TF), lambda i: (i, 0))
    ospec = pl.BlockSpec((TF, bp), lambda i: (0, i))
    return pl.pallas_call(
        _combine_body,
        grid=(nblk,),
        in_specs=[wspec] * 4 + [rspec] * 4,
        out_specs=ospec,
        out_shape=jax.ShapeDtypeStruct((TF, B), jnp.float32),
    )(w00, w01, w10, w11, r00, r01, r10, r11)


def kernel(expressions, audio_features, uv_inputs, data):
    x = uv_inputs[0, 0]
    y = uv_inputs[0, 1]
    table = jnp.transpose(data[0].reshape(TF, TD * TD), (1, 0))

    i00, i01, i10, i11, w00, w01, w10, w11 = _prep(x, y)
    r00, r01, r10, r11 = _gather(
        table,
        i00.reshape(B), i01.reshape(B), i10.reshape(B), i11.reshape(B))
    out = _combine(
        w00.reshape(1, B), w01.reshape(1, B), w10.reshape(1, B),
        w11.reshape(1, B), r00, r01, r10, r11)
    return out.reshape(1, TF, HO, WO)


# speedup vs baseline: 1.6992x; 1.6992x over previous
import functools

import jax
import jax.numpy as jnp
from jax import lax
from jax.experimental import pallas as pl
from jax.experimental.pallas import tpu as pltpu
from jax.experimental.pallas import tpu_sc as plsc

TD = 1024
TF = 16
HO = 512
WO = 512
B = HO * WO
NC = 2
NS = 16
NW = NC * NS
PIX_PER_W = B // NW
CHUNK = 1024
N_CHUNKS = PIX_PER_W // CHUNK


def _prep_body(x_ref, y_ref, i00_ref, i01_ref, i10_ref, i11_ref,
               w00_ref, w01_ref, w10_ref, w11_ref):
    x = x_ref[...]
    y = y_ref[...]
    ix = jnp.clip(((x + 1.0) * TD - 1.0) * 0.5, 0.0, TD - 1.0)
    iy = jnp.clip(((y + 1.0) * TD - 1.0) * 0.5, 0.0, TD - 1.0)
    ix0 = jnp.floor(ix)
    iy0 = jnp.floor(iy)
    wx1 = ix - ix0
    wx0 = 1.0 - wx1
    wy1 = iy - iy0
    wy0 = 1.0 - wy1
    x0 = ix0.astype(jnp.int32)
    y0 = iy0.astype(jnp.int32)
    y1 = jnp.minimum(y0 + 1, TD - 1)
    x1 = jnp.minimum(x0 + 1, TD - 1)
    i00_ref[...] = y0 * TD + x0
    i01_ref[...] = y0 * TD + x1
    i10_ref[...] = y1 * TD + x0
    i11_ref[...] = y1 * TD + x1
    w00_ref[...] = wy0 * wx0
    w01_ref[...] = wy0 * wx1
    w10_ref[...] = wy1 * wx0
    w11_ref[...] = wy1 * wx1


def _prep(x, y):
    blk = pl.BlockSpec((64, WO), lambda i: (i, 0))
    shp_i = jax.ShapeDtypeStruct((HO, WO), jnp.int32)
    shp_f = jax.ShapeDtypeStruct((HO, WO), jnp.float32)
    return pl.pallas_call(
        _prep_body,
        grid=(HO // 64,),
        in_specs=[blk, blk],
        out_specs=[blk] * 8,
        out_shape=[shp_i] * 4 + [shp_f] * 4,
    )(x, y)


def _sc_body(table, i00, i01, i10, i11, w00, w01, w10, w11, out_hbm,
             iv0, iv1, iv2, iv3, r0, r1, r2, r3,
             wv0, wv1, wv2, wv3, out_v, sem):
    wid = lax.axis_index("s") * NC + lax.axis_index("c")
    base = wid * PIX_PER_W
    lane_const = [jnp.full((16, 1), j, jnp.int32) for j in range(16)]
    bcast_dn = lax.GatherDimensionNumbers(
        offset_dims=(), collapsed_slice_dims=(0,), start_index_map=(0,))

    def bcast(vec, j):
        return lax.gather(vec, lane_const[j], bcast_dn, slice_sizes=(1,),
                          mode=lax.GatherScatterMode.PROMISE_IN_BOUNDS)

    for h in range(N_CHUNKS):
        off = base + h * CHUNK
        pltpu.sync_copy(i00.at[pl.ds(off, CHUNK)], iv0)
        pltpu.sync_copy(i01.at[pl.ds(off, CHUNK)], iv1)
        pltpu.sync_copy(i10.at[pl.ds(off, CHUNK)], iv2)
        pltpu.sync_copy(i11.at[pl.ds(off, CHUNK)], iv3)
        c0 = pltpu.async_copy(table.at[iv0], r0, sem)
        c1 = pltpu.async_copy(table.at[iv1], r1, sem)
        c2 = pltpu.async_copy(table.at[iv2], r2, sem)
        c3 = pltpu.async_copy(table.at[iv3], r3, sem)
        pltpu.sync_copy(w00.at[pl.ds(off, CHUNK)], wv0)
        pltpu.sync_copy(w01.at[pl.ds(off, CHUNK)], wv1)
        pltpu.sync_copy(w10.at[pl.ds(off, CHUNK)], wv2)
        pltpu.sync_copy(w11.at[pl.ds(off, CHUNK)], wv3)
        c0.wait()
        c1.wait()
        c2.wait()
        c3.wait()

        def grp_body(g, carry):
            p0 = g * 16
            wg0 = wv0[pl.ds(p0, 16)]
            wg1 = wv1[pl.ds(p0, 16)]
            wg2 = wv2[pl.ds(p0, 16)]
            wg3 = wv3[pl.ds(p0, 16)]
            for j in range(16):
                p = p0 + j
                b0 = bcast(wg0, j)
                b1 = bcast(wg1, j)
                b2 = bcast(wg2, j)
                b3 = bcast(wg3, j)
                acc = r0[p] * b0 + r1[p] * b1 + r2[p] * b2 + r3[p] * b3
                out_v[pl.ds(p * 16, 16)] = acc
            return carry

        lax.fori_loop(0, CHUNK // 16, grp_body, 0)
        pltpu.sync_copy(out_v, out_hbm.at[pl.ds(off * 16, CHUNK * 16)])


def _sc_gather_combine(table, i00, i01, i10, i11, w00, w01, w10, w11):
    mesh = plsc.VectorSubcoreMesh(core_axis_name="c", subcore_axis_name="s")
    k = functools.partial(
        pl.kernel,
        mesh=mesh,
        compiler_params=pltpu.CompilerParams(use_tc_tiling_on_sc=False),
        out_type=jax.ShapeDtypeStruct((B * TF,), jnp.float32),
        scratch_types=[
            pltpu.VMEM((CHUNK,), jnp.int32),
            pltpu.VMEM((CHUNK,), jnp.int32),
            pltpu.VMEM((CHUNK,), jnp.int32),
            pltpu.VMEM((CHUNK,), jnp.int32),
            pltpu.VMEM((CHUNK, TF), jnp.float32),
            pltpu.VMEM((CHUNK, TF), jnp.float32),
            pltpu.VMEM((CHUNK, TF), jnp.float32),
            pltpu.VMEM((CHUNK, TF), jnp.float32),
            pltpu.VMEM((CHUNK,), jnp.float32),
            pltpu.VMEM((CHUNK,), jnp.float32),
            pltpu.VMEM((CHUNK,), jnp.float32),
            pltpu.VMEM((CHUNK,), jnp.float32),
            pltpu.VMEM((CHUNK * TF,), jnp.float32),
            pltpu.SemaphoreType.DMA,
        ],
    )(_sc_body)
    return k(table, i00, i01, i10, i11, w00, w01, w10, w11)


def kernel(expressions, audio_features, uv_inputs, data):
    x = uv_inputs[0, 0]
    y = uv_inputs[0, 1]
    table = jnp.transpose(data[0].reshape(TF, TD * TD), (1, 0))

    i00, i01, i10, i11, w00, w01, w10, w11 = _prep(x, y)
    out_flat = _sc_gather_combine(
        table,
        i00.reshape(B), i01.reshape(B), i10.reshape(B), i11.reshape(B),
        w00.reshape(B), w01.reshape(B), w10.reshape(B), w11.reshape(B))
    out = jnp.transpose(out_flat.reshape(B, TF), (1, 0))
    return out.reshape(1, TF, HO, WO)


# speedup vs baseline: 2.9338x; 1.7266x over previous
import functools

import jax
import jax.numpy as jnp
import numpy as np
from jax import lax
from jax.experimental import pallas as pl
from jax.experimental.pallas import tpu as pltpu
from jax.experimental.pallas import tpu_sc as plsc

TD = 1024
TF = 16
HO = 512
WO = 512
B = HO * WO
NC = 2
NS = 16
NW = NC * NS
PIX_PER_W = B // NW
CHUNK = 1024
N_CHUNKS = PIX_PER_W // CHUNK


def _tx_body(x_ref, o_ref):
    r = lax.broadcasted_iota(jnp.int32, (128, 128), 0)
    o = lax.broadcasted_iota(jnp.int32, (128, 128), 1)
    p = ((r % 8) * 16 + r // 8 == o).astype(jnp.float32)
    x = x_ref[...].reshape(128, TD)
    o_ref[...] = lax.dot_general(
        x, p, (((0,), (0,)), ((), ())),
        preferred_element_type=jnp.float32)


def _tx(d):
    return pl.pallas_call(
        _tx_body,
        grid=(TD // 8,),
        in_specs=[pl.BlockSpec((TF, 8, TD), lambda i: (0, i, 0))],
        out_specs=pl.BlockSpec((TD, 128), lambda i: (i, 0)),
        out_shape=jax.ShapeDtypeStruct((TD * TD // 8, 128), jnp.float32),
    )(d)


def _prep_body(x_ref, y_ref, i00_ref, i01_ref, i10_ref, i11_ref,
               w00_ref, w01_ref, w10_ref, w11_ref):
    x = x_ref[...]
    y = y_ref[...]
    ix = jnp.clip(((x + 1.0) * TD - 1.0) * 0.5, 0.0, TD - 1.0)
    iy = jnp.clip(((y + 1.0) * TD - 1.0) * 0.5, 0.0, TD - 1.0)
    ix0 = jnp.floor(ix)
    iy0 = jnp.floor(iy)
    wx1 = ix - ix0
    wx0 = 1.0 - wx1
    wy1 = iy - iy0
    wy0 = 1.0 - wy1
    x0 = ix0.astype(jnp.int32)
    y0 = iy0.astype(jnp.int32)
    y1 = jnp.minimum(y0 + 1, TD - 1)
    x1 = jnp.minimum(x0 + 1, TD - 1)
    yb0 = (y0 >> 3) * 8192 + (y0 & 7)
    yb1 = (y1 >> 3) * 8192 + (y1 & 7)
    x0_8 = x0 * 8
    x1_8 = x1 * 8
    i00_ref[...] = yb0 + x0_8
    i01_ref[...] = yb0 + x1_8
    i10_ref[...] = yb1 + x0_8
    i11_ref[...] = yb1 + x1_8
    w00_ref[...] = wy0 * wx0
    w01_ref[...] = wy0 * wx1
    w10_ref[...] = wy1 * wx0
    w11_ref[...] = wy1 * wx1


def _prep(x, y):
    blk = pl.BlockSpec((64, WO), lambda i: (i, 0))
    shp_i = jax.ShapeDtypeStruct((HO, WO), jnp.int32)
    shp_f = jax.ShapeDtypeStruct((HO, WO), jnp.float32)
    return pl.pallas_call(
        _prep_body,
        grid=(HO // 64,),
        in_specs=[blk, blk],
        out_specs=[blk] * 8,
        out_shape=[shp_i] * 4 + [shp_f] * 4,
    )(x, y)


def _sc_body(table, i00, i01, i10, i11, w00, w01, w10, w11, out_hbm,
             iv0, iv1, iv2, iv3, r0, r1, r2, r3,
             wv0, wv1, wv2, wv3, out_v, sem):
    wid = lax.axis_index("s") * NC + lax.axis_index("c")
    base = wid * PIX_PER_W
    lane_const = [jnp.full((16, 1), j, jnp.int32) for j in range(16)]
    bcast_dn = lax.GatherDimensionNumbers(
        offset_dims=(), collapsed_slice_dims=(0,), start_index_map=(0,))

    def bcast(vec, j):
        return lax.gather(vec, lane_const[j], bcast_dn, slice_sizes=(1,),
                          mode=lax.GatherScatterMode.PROMISE_IN_BOUNDS)

    for h in range(N_CHUNKS):
        off = base + h * CHUNK
        pltpu.sync_copy(i00.at[pl.ds(off, CHUNK)], iv0)
        pltpu.sync_copy(i01.at[pl.ds(off, CHUNK)], iv1)
        pltpu.sync_copy(i10.at[pl.ds(off, CHUNK)], iv2)
        pltpu.sync_copy(i11.at[pl.ds(off, CHUNK)], iv3)
        c0 = pltpu.async_copy(table.at[iv0], r0, sem)
        c1 = pltpu.async_copy(table.at[iv1], r1, sem)
        c2 = pltpu.async_copy(table.at[iv2], r2, sem)
        c3 = pltpu.async_copy(table.at[iv3], r3, sem)
        pltpu.sync_copy(w00.at[pl.ds(off, CHUNK)], wv0)
        pltpu.sync_copy(w01.at[pl.ds(off, CHUNK)], wv1)
        pltpu.sync_copy(w10.at[pl.ds(off, CHUNK)], wv2)
        pltpu.sync_copy(w11.at[pl.ds(off, CHUNK)], wv3)
        c0.wait()
        c1.wait()
        c2.wait()
        c3.wait()

        def grp_body(g, carry):
            p0 = g * 16
            wg0 = wv0[pl.ds(p0, 16)]
            wg1 = wv1[pl.ds(p0, 16)]
            wg2 = wv2[pl.ds(p0, 16)]
            wg3 = wv3[pl.ds(p0, 16)]
            for j in range(16):
                p = p0 + j
                b0 = bcast(wg0, j)
                b1 = bcast(wg1, j)
                b2 = bcast(wg2, j)
                b3 = bcast(wg3, j)
                acc = r0[p] * b0 + r1[p] * b1 + r2[p] * b2 + r3[p] * b3
                out_v[pl.ds(p * 16, 16)] = acc
            return carry

        lax.fori_loop(0, CHUNK // 16, grp_body, 0)
        pltpu.sync_copy(out_v, out_hbm.at[pl.ds(off * 16, CHUNK * 16)])


def _sc_gather_combine(table, i00, i01, i10, i11, w00, w01, w10, w11):
    mesh = plsc.VectorSubcoreMesh(core_axis_name="c", subcore_axis_name="s")
    k = functools.partial(
        pl.kernel,
        mesh=mesh,
        compiler_params=pltpu.CompilerParams(use_tc_tiling_on_sc=False),
        out_type=jax.ShapeDtypeStruct((B * TF,), jnp.float32),
        scratch_types=[
            pltpu.VMEM((CHUNK,), jnp.int32),
            pltpu.VMEM((CHUNK,), jnp.int32),
            pltpu.VMEM((CHUNK,), jnp.int32),
            pltpu.VMEM((CHUNK,), jnp.int32),
            pltpu.VMEM((CHUNK, TF), jnp.float32),
            pltpu.VMEM((CHUNK, TF), jnp.float32),
            pltpu.VMEM((CHUNK, TF), jnp.float32),
            pltpu.VMEM((CHUNK, TF), jnp.float32),
            pltpu.VMEM((CHUNK,), jnp.float32),
            pltpu.VMEM((CHUNK,), jnp.float32),
            pltpu.VMEM((CHUNK,), jnp.float32),
            pltpu.VMEM((CHUNK,), jnp.float32),
            pltpu.VMEM((CHUNK * TF,), jnp.float32),
            pltpu.SemaphoreType.DMA,
        ],
    )(_sc_body)
    return k(table, i00, i01, i10, i11, w00, w01, w10, w11)


def kernel(expressions, audio_features, uv_inputs, data):
    x = uv_inputs[0, 0]
    y = uv_inputs[0, 1]
    table = _tx(data[0]).reshape(TD * TD, TF)

    i00, i01, i10, i11, w00, w01, w10, w11 = _prep(x, y)
    out_flat = _sc_gather_combine(
        table,
        i00.reshape(B), i01.reshape(B), i10.reshape(B), i11.reshape(B),
        w00.reshape(B), w01.reshape(B), w10.reshape(B), w11.reshape(B))
    out = jnp.transpose(out_flat.reshape(B, TF), (1, 0))
    return out.reshape(1, TF, HO, WO)
